# baseline (speedup 1.0000x reference)
import functools

import jax
import jax.numpy as jnp
from jax.experimental import pallas as pl
from jax.experimental.pallas import tpu as pltpu

B, T = 1024, 50
V = 1000
BB = 16
GRID = B // BB


def _body(x_ref, hi_ref, lo_ref, out_ref):
    hi = hi_ref[...]
    lo = lo_ref[...]
    cols = jax.lax.broadcasted_iota(jnp.int32, (T, V), 1)
    for i in range(BB):
        idx = x_ref[i, :].reshape(T, 1)
        onehot = jnp.where(cols == idx, 1.0, 0.0).astype(jnp.bfloat16)
        acc = jnp.dot(onehot, hi, preferred_element_type=jnp.float32)
        acc += jnp.dot(onehot, lo, preferred_element_type=jnp.float32)
        out_ref[i] = acc


@functools.partial(jax.jit)
def kernel(x, logits_table):
    hi = logits_table.astype(jnp.bfloat16)
    lo = (logits_table - hi.astype(jnp.float32)).astype(jnp.bfloat16)
    return pl.pallas_call(
        _body,
        grid=(GRID,),
        in_specs=[
            pl.BlockSpec((BB, T), lambda m: (m, 0)),
            pl.BlockSpec((V, V), lambda m: (0, 0)),
            pl.BlockSpec((V, V), lambda m: (0, 0)),
        ],
        out_specs=pl.BlockSpec((BB, T, V), lambda m: (m, 0, 0)),
        out_shape=jax.ShapeDtypeStruct((B, T, V), jnp.float32),
        compiler_params=pltpu.CompilerParams(
            dimension_semantics=("arbitrary",),
        ),
    )(x.astype(jnp.int32), hi, lo)


# speedup vs baseline: 1.2488x; 1.2488x over previous
import functools

import jax
import jax.numpy as jnp
from jax import lax
from jax.experimental import pallas as pl
from jax.experimental.pallas import tpu as pltpu
from jax.experimental.pallas import tpu_sc as plsc

B, T = 1024, 50
TP = 56
ROW = 1000
ROWP = 1024
NUM_WORKERS = 32
PER_WORKER = B // NUM_WORKERS

_MESH = plsc.VectorSubcoreMesh(core_axis_name="c", subcore_axis_name="s")


@functools.partial(
    pl.kernel,
    mesh=_MESH,
    out_type=jax.ShapeDtypeStruct((B, TP, ROWP), jnp.float32),
    scratch_types=[
        pltpu.VMEM((TP,), jnp.int32),
        pltpu.VMEM((TP, ROWP), jnp.float32),
        pltpu.SemaphoreType.DMA,
    ],
)
def _gather(idx_hbm, table_hbm, out_hbm, idx_v, rows_v, sem):
    wid = lax.axis_index("s") * 2 + lax.axis_index("c")
    base = wid * PER_WORKER

    def body(j, carry):
        b = base + j
        pltpu.sync_copy(idx_hbm.at[b], idx_v)
        pltpu.async_copy(table_hbm.at[idx_v], rows_v, sem).wait()
        pltpu.sync_copy(rows_v, out_hbm.at[b])
        return carry

    lax.fori_loop(0, PER_WORKER, body, 0)


def kernel(x, logits_table):
    idx = jnp.pad(x.astype(jnp.int32), ((0, 0), (0, TP - T)))
    tpad = jnp.pad(logits_table, ((0, 0), (0, ROWP - ROW)))
    out = _gather(idx, tpad)
    return out[:, :T, :ROW]


# speedup vs baseline: 1.2541x; 1.0042x over previous
import functools

import jax
import jax.numpy as jnp
from jax import lax
from jax.experimental import pallas as pl
from jax.experimental.pallas import tpu as pltpu
from jax.experimental.pallas import tpu_sc as plsc

B, T = 1024, 50
TP = 56
ROW = 1000
ROWP = 1024
UNITS = TP * ROWP // 128
NSPLIT = 4
USPLIT = UNITS // NSPLIT
NUM_WORKERS = 32
PER_WORKER = B // NUM_WORKERS

_MESH = plsc.VectorSubcoreMesh(core_axis_name="c", subcore_axis_name="s")


@functools.partial(
    pl.kernel,
    mesh=_MESH,
    out_type=jax.ShapeDtypeStruct((B, TP, ROWP), jnp.float32),
    scratch_types=[
        pltpu.VMEM((NSPLIT, USPLIT), jnp.int32),
        pltpu.VMEM((UNITS, 128), jnp.float32),
        pltpu.SemaphoreType.DMA,
    ],
)
def _gather(idx_hbm, table_hbm, out_hbm, idx_v, units_v, sem):
    wid = lax.axis_index("s") * 2 + lax.axis_index("c")
    base = wid * PER_WORKER

    def body(j, carry):
        b = base + j
        pltpu.sync_copy(idx_hbm.at[b], idx_v)
        for s in range(NSPLIT):
            pltpu.async_copy(
                table_hbm.at[idx_v.at[s]],
                units_v.at[pl.ds(s * USPLIT, USPLIT)],
                sem,
            )
        for s in range(NSPLIT):
            pltpu.make_async_copy(
                table_hbm.at[idx_v.at[s]],
                units_v.at[pl.ds(s * USPLIT, USPLIT)],
                sem,
            ).wait()
        pltpu.sync_copy(units_v.reshape(TP, ROWP), out_hbm.at[b])
        return carry

    lax.fori_loop(0, PER_WORKER, body, 0)


def kernel(x, logits_table):
    xp = jnp.pad(x.astype(jnp.int32), ((0, 0), (0, TP - T)))
    k = jnp.arange(UNITS)
    t_k = 8 * (k // 64) + k % 8
    j_k = (k // 8) % 8
    idx8 = (xp[:, t_k] * 8 + j_k[None, :]).reshape(B, NSPLIT, USPLIT)
    table8 = jnp.pad(logits_table, ((0, 0), (0, ROWP - ROW))).reshape(-1, 128)
    out = _gather(idx8.astype(jnp.int32), table8)
    return out[:, :T, :ROW]
